# baseline (speedup 1.0000x reference)
import functools

import jax
import jax.numpy as jnp
from jax import lax
from jax.experimental import pallas as pl
from jax.experimental.pallas import tpu as pltpu
from jax.experimental.pallas import tpu_sc as plsc

NE = 64
S = 2048
D = 768
HD = 768
BT = 256
NT = S // BT
MAXV = NT + NE
NW = 32
CHUNK = S // NW


def _router_body(x_ref, rw_ref, rb_ref, p_ref, gid_ref, tid_ref,
                 lo_ref, hi_ref):
    x = x_ref[...]
    rw = rw_ref[...]
    logits = lax.dot_general(x, rw, (((1,), (1,)), ((), ())),
                             preferred_element_type=jnp.float32)
    logits = logits + rb_ref[...]
    m = jnp.max(logits, axis=1, keepdims=True)
    iota_e = lax.broadcasted_iota(jnp.int32, (S, NE), 1)
    e = jnp.min(jnp.where(logits == m, iota_e, NE), axis=1, keepdims=True)
    onehot = (iota_e == e).astype(jnp.float32)
    counts = jnp.sum(onehot, axis=0, keepdims=True)
    ia = lax.broadcasted_iota(jnp.int32, (NE, NE), 0)
    ib = lax.broadcasted_iota(jnp.int32, (NE, NE), 1)
    lt = (ia < ib).astype(jnp.float32)
    offs = lax.dot_general(counts, lt, (((1,), (0,)), ((), ())))
    blk = 256
    run = jnp.zeros((1, NE), jnp.float32)
    parts = []
    ii = lax.broadcasted_iota(jnp.int32, (blk, blk), 0)
    jj = lax.broadcasted_iota(jnp.int32, (blk, blk), 1)
    tril = (ii > jj).astype(jnp.float32)
    for b in range(S // blk):
        hb = onehot[b * blk:(b + 1) * blk]
        cb = lax.dot_general(tril, hb, (((1,), (0,)), ((), ()))) + run
        run = run + jnp.sum(hb, axis=0, keepdims=True)
        parts.append(cb)
    rank = jnp.concatenate(parts, axis=0)
    p = jnp.sum(onehot * (rank + offs), axis=1, keepdims=True)
    p_ref[...] = p.astype(jnp.int32).reshape(S // 128, 128)

    end = offs + counts
    nonempty = counts > 0.0
    first_tile = jnp.floor(offs / BT)
    last_tile = jnp.where(nonempty, jnp.floor((end - 1.0) / BT), 0.0)
    gtiles = jnp.where(nonempty, last_tile - first_tile + 1.0, 0.0)
    le = (ia <= ib).astype(jnp.float32)
    vs = lax.dot_general(gtiles, le, (((1,), (0,)), ((), ())))
    nvisit = jnp.max(vs)
    t_col = lax.broadcasted_iota(jnp.int32, (MAXV, 1), 0).astype(jnp.float32)
    gid = jnp.sum((vs <= t_col).astype(jnp.float32), axis=1, keepdims=True)
    valid = t_col < nvisit
    last_gid = jnp.sum((vs <= nvisit - 1.0).astype(jnp.float32))
    gid_c = jnp.where(valid, gid, last_gid)
    iota_g = lax.broadcasted_iota(jnp.int32, (MAXV, NE), 1).astype(jnp.float32)
    onehot_g = (iota_g == gid_c).astype(jnp.float32)

    def sel(v):
        return lax.dot_general(onehot_g, v, (((1,), (1,)), ((), ())))

    vstart = sel(vs - gtiles)
    tid = jnp.where(valid, sel(first_tile) + (t_col - vstart), float(NT - 1))
    gid_ref[...] = gid_c.astype(jnp.int32)
    tid_ref[...] = tid.astype(jnp.int32)
    lo_ref[...] = jnp.where(valid, sel(offs), 0.0).astype(jnp.int32)
    hi_ref[...] = jnp.where(valid, sel(end), 0.0).astype(jnp.int32)


_router = pl.pallas_call(
    _router_body,
    out_shape=(
        jax.ShapeDtypeStruct((S // 128, 128), jnp.int32),
        jax.ShapeDtypeStruct((MAXV, 1), jnp.int32),
        jax.ShapeDtypeStruct((MAXV, 1), jnp.int32),
        jax.ShapeDtypeStruct((MAXV, 1), jnp.int32),
        jax.ShapeDtypeStruct((MAXV, 1), jnp.int32),
    ),
)


HD2 = HD // 2


def _gmm_body(gid_ref, tid_ref, lo_ref, hi_ref,
              xs_ref, wa_ref, wb_ref, b_ref, o_ref):
    t = pl.program_id(0)
    lo = lo_ref[t, 0]
    hi = hi_ref[t, 0]
    tile = tid_ref[t, 0]

    @pl.when(lo < hi)
    def _():
        g = gid_ref[t, 0]
        xb = xs_ref[...]
        r = tile * BT + lax.broadcasted_iota(jnp.int32, (BT, HD2), 0)
        mask = (r >= lo) & (r < hi)
        acc0 = lax.dot_general(xb, wa_ref[0], (((1,), (1,)), ((), ())),
                               precision=lax.Precision.DEFAULT,
                               preferred_element_type=jnp.float32)
        acc0 = acc0 + b_ref[pl.ds(g, 1), 0:HD2]
        o_ref[:, 0:HD2] = jnp.where(mask, acc0, o_ref[:, 0:HD2])
        acc1 = lax.dot_general(xb, wb_ref[0], (((1,), (1,)), ((), ())),
                               precision=lax.Precision.DEFAULT,
                               preferred_element_type=jnp.float32)
        acc1 = acc1 + b_ref[pl.ds(g, 1), HD2:HD]
        o_ref[:, HD2:HD] = jnp.where(mask, acc1, o_ref[:, HD2:HD])


_gmm = pl.pallas_call(
    _gmm_body,
    grid_spec=pltpu.PrefetchScalarGridSpec(
        num_scalar_prefetch=4,
        grid=(MAXV,),
        in_specs=[
            pl.BlockSpec((BT, D), lambda t, gid, tid, lo, hi: (tid[t, 0], 0)),
            pl.BlockSpec((1, HD2, D),
                         lambda t, gid, tid, lo, hi: (gid[t, 0], 0, 0)),
            pl.BlockSpec((1, HD2, D),
                         lambda t, gid, tid, lo, hi: (gid[t, 0], 1, 0)),
            pl.BlockSpec((NE, HD), lambda t, gid, tid, lo, hi: (0, 0)),
        ],
        out_specs=pl.BlockSpec((BT, HD),
                               lambda t, gid, tid, lo, hi: (tid[t, 0], 0)),
    ),
    out_shape=jax.ShapeDtypeStruct((S, HD), jnp.float32),
    compiler_params=pltpu.CompilerParams(dimension_semantics=("arbitrary",)),
)


@functools.cache
def _sc_kernels():
    mesh = plsc.VectorSubcoreMesh(core_axis_name="c", subcore_axis_name="s")

    @functools.partial(
        pl.kernel,
        out_type=jax.ShapeDtypeStruct((S, D), jnp.float32),
        mesh=mesh,
        scratch_types=[
            pltpu.VMEM((CHUNK,), jnp.int32),
            pltpu.VMEM((CHUNK, D), jnp.float32),
            pltpu.SemaphoreType.DMA,
        ],
    )
    def sc_scatter_rows(x_hbm, p_hbm, out_hbm, idx_v, rows_v, sem):
        wid = lax.axis_index("s") * 2 + lax.axis_index("c")
        base = wid * CHUNK
        pltpu.sync_copy(p_hbm.at[wid // 2, pl.ds((wid % 2) * CHUNK, CHUNK)],
                        idx_v)
        pltpu.sync_copy(x_hbm.at[pl.ds(base, CHUNK)], rows_v)
        pltpu.async_copy(rows_v, out_hbm.at[idx_v], sem).wait()

    @functools.partial(
        pl.kernel,
        out_type=jax.ShapeDtypeStruct((S, HD), jnp.float32),
        mesh=mesh,
        scratch_types=[
            pltpu.VMEM((CHUNK,), jnp.int32),
            pltpu.VMEM((CHUNK, HD), jnp.float32),
            pltpu.SemaphoreType.DMA,
        ],
    )
    def sc_gather_rows(ys_hbm, p_hbm, out_hbm, idx_v, rows_v, sem):
        wid = lax.axis_index("s") * 2 + lax.axis_index("c")
        base = wid * CHUNK
        pltpu.sync_copy(p_hbm.at[wid // 2, pl.ds((wid % 2) * CHUNK, CHUNK)],
                        idx_v)
        pltpu.async_copy(ys_hbm.at[idx_v], rows_v, sem).wait()
        pltpu.sync_copy(rows_v, out_hbm.at[pl.ds(base, CHUNK)])

    return sc_scatter_rows, sc_gather_rows


def kernel(x, router_w, router_b, expert_w, expert_b):
    b, s, d = x.shape
    x_flat = x.reshape(s, d)
    p, gid, tid, lo, hi = _router(x_flat, router_w, router_b.reshape(1, NE))
    sc_scatter_rows, sc_gather_rows = _sc_kernels()
    xs = sc_scatter_rows(x_flat, p)
    ys = _gmm(gid, tid, lo, hi, xs, expert_w, expert_w, expert_b)
    out = sc_gather_rows(ys, p)
    return out.reshape(b, s, HD)


# speedup vs baseline: 1.0169x; 1.0169x over previous
import functools

import jax
import jax.numpy as jnp
from jax import lax
from jax.experimental import pallas as pl
from jax.experimental.pallas import tpu as pltpu
from jax.experimental.pallas import tpu_sc as plsc

NE = 64
S = 2048
D = 768
HD = 768
BT = 256
NT = S // BT
MAXV = NT + NE
NW = 32
CHUNK = S // NW


def _router_body(x_ref, rw_ref, rb_ref, p_ref, gid_ref, tid_ref,
                 lo_ref, hi_ref):
    x = x_ref[...]
    rw = rw_ref[...]
    logits = lax.dot_general(x, rw, (((1,), (1,)), ((), ())),
                             preferred_element_type=jnp.float32)
    logits = logits + rb_ref[...]
    m = jnp.max(logits, axis=1, keepdims=True)
    iota_e = lax.broadcasted_iota(jnp.int32, (S, NE), 1)
    e = jnp.min(jnp.where(logits == m, iota_e, NE), axis=1, keepdims=True)
    onehot = (iota_e == e).astype(jnp.float32)
    counts = jnp.sum(onehot, axis=0, keepdims=True)
    ia = lax.broadcasted_iota(jnp.int32, (NE, NE), 0)
    ib = lax.broadcasted_iota(jnp.int32, (NE, NE), 1)
    lt = (ia < ib).astype(jnp.float32)
    offs = lax.dot_general(counts, lt, (((1,), (0,)), ((), ())))
    blk = 256
    run = jnp.zeros((1, NE), jnp.float32)
    parts = []
    ii = lax.broadcasted_iota(jnp.int32, (blk, blk), 0)
    jj = lax.broadcasted_iota(jnp.int32, (blk, blk), 1)
    tril = (ii > jj).astype(jnp.float32)
    for b in range(S // blk):
        hb = onehot[b * blk:(b + 1) * blk]
        cb = lax.dot_general(tril, hb, (((1,), (0,)), ((), ()))) + run
        run = run + jnp.sum(hb, axis=0, keepdims=True)
        parts.append(cb)
    rank = jnp.concatenate(parts, axis=0)
    p = jnp.sum(onehot * (rank + offs), axis=1, keepdims=True)
    p_ref[...] = p.astype(jnp.int32).reshape(S // 128, 128)

    end = offs + counts
    nonempty = counts > 0.0
    first_tile = jnp.floor(offs / BT)
    last_tile = jnp.where(nonempty, jnp.floor((end - 1.0) / BT), 0.0)
    gtiles = jnp.where(nonempty, last_tile - first_tile + 1.0, 0.0)
    le = (ia <= ib).astype(jnp.float32)
    vs = lax.dot_general(gtiles, le, (((1,), (0,)), ((), ())))
    nvisit = jnp.max(vs)
    t_col = lax.broadcasted_iota(jnp.int32, (MAXV, 1), 0).astype(jnp.float32)
    gid = jnp.sum((vs <= t_col).astype(jnp.float32), axis=1, keepdims=True)
    valid = t_col < nvisit
    last_gid = jnp.sum((vs <= nvisit - 1.0).astype(jnp.float32))
    gid_c = jnp.where(valid, gid, last_gid)
    iota_g = lax.broadcasted_iota(jnp.int32, (MAXV, NE), 1).astype(jnp.float32)
    onehot_g = (iota_g == gid_c).astype(jnp.float32)

    def sel(v):
        return lax.dot_general(onehot_g, v, (((1,), (1,)), ((), ())))

    vstart = sel(vs - gtiles)
    tid = jnp.where(valid, sel(first_tile) + (t_col - vstart), float(NT - 1))
    gid_ref[...] = gid_c.astype(jnp.int32)
    tid_ref[...] = tid.astype(jnp.int32)
    lo_ref[...] = jnp.where(valid, sel(offs), 0.0).astype(jnp.int32)
    hi_ref[...] = jnp.where(valid, sel(end), 0.0).astype(jnp.int32)


_router = pl.pallas_call(
    _router_body,
    out_shape=(
        jax.ShapeDtypeStruct((S // 128, 128), jnp.int32),
        jax.ShapeDtypeStruct((MAXV, 1), jnp.int32),
        jax.ShapeDtypeStruct((MAXV, 1), jnp.int32),
        jax.ShapeDtypeStruct((MAXV, 1), jnp.int32),
        jax.ShapeDtypeStruct((MAXV, 1), jnp.int32),
    ),
)


def _gmm_body(gid_ref, tid_ref, lo_ref, hi_ref,
              xs_ref, wa_ref, b_ref, o_ref):
    t = pl.program_id(0)
    lo = lo_ref[t, 0]
    hi = hi_ref[t, 0]
    tile = tid_ref[t, 0]

    @pl.when(lo < hi)
    def _():
        acc = lax.dot_general(xs_ref[...], wa_ref[0],
                              (((1,), (1,)), ((), ())),
                              precision=lax.Precision.DEFAULT,
                              preferred_element_type=jnp.float32)
        acc = acc + b_ref[pl.ds(gid_ref[t, 0], 1), :]
        r = tile * BT + lax.broadcasted_iota(jnp.int32, (BT, HD), 0)
        mask = (r >= lo) & (r < hi)
        o_ref[...] = jnp.where(mask, acc, o_ref[...])


_gmm = pl.pallas_call(
    _gmm_body,
    grid_spec=pltpu.PrefetchScalarGridSpec(
        num_scalar_prefetch=4,
        grid=(MAXV,),
        in_specs=[
            pl.BlockSpec((BT, D), lambda t, gid, tid, lo, hi: (tid[t, 0], 0)),
            pl.BlockSpec((1, HD, D),
                         lambda t, gid, tid, lo, hi: (gid[t, 0], 0, 0)),
            pl.BlockSpec((NE, HD), lambda t, gid, tid, lo, hi: (0, 0)),
        ],
        out_specs=pl.BlockSpec((BT, HD),
                               lambda t, gid, tid, lo, hi: (tid[t, 0], 0)),
    ),
    out_shape=jax.ShapeDtypeStruct((S, HD), jnp.float32),
    compiler_params=pltpu.CompilerParams(dimension_semantics=("arbitrary",)),
)


HC = CHUNK // 2


@functools.cache
def _sc_kernels():
    mesh = plsc.VectorSubcoreMesh(core_axis_name="c", subcore_axis_name="s")

    @functools.partial(
        pl.kernel,
        out_type=jax.ShapeDtypeStruct((S, D), jnp.float32),
        mesh=mesh,
        scratch_types=[
            pltpu.VMEM((HC,), jnp.int32),
            pltpu.VMEM((HC,), jnp.int32),
            pltpu.VMEM((HC, D), jnp.float32),
            pltpu.VMEM((HC, D), jnp.float32),
            pltpu.SemaphoreType.DMA,
            pltpu.SemaphoreType.DMA,
            pltpu.SemaphoreType.DMA,
        ],
    )
    def sc_scatter_rows(x_hbm, p_hbm, out_hbm,
                        idx0, idx1, r0, r1, s0, s1, ss):
        wid = lax.axis_index("s") * 2 + lax.axis_index("c")
        base = wid * CHUNK
        row = wid // 2
        col = (wid % 2) * CHUNK
        c0 = pltpu.async_copy(x_hbm.at[pl.ds(base, HC)], r0, s0)
        c1 = pltpu.async_copy(x_hbm.at[pl.ds(base + HC, HC)], r1, s1)
        pltpu.sync_copy(p_hbm.at[row, pl.ds(col, HC)], idx0)
        pltpu.sync_copy(p_hbm.at[row, pl.ds(col + HC, HC)], idx1)
        c0.wait()
        w0 = pltpu.async_copy(r0, out_hbm.at[idx0], ss)
        c1.wait()
        w1 = pltpu.async_copy(r1, out_hbm.at[idx1], ss)
        w0.wait()
        w1.wait()

    @functools.partial(
        pl.kernel,
        out_type=jax.ShapeDtypeStruct((S, HD), jnp.float32),
        mesh=mesh,
        scratch_types=[
            pltpu.VMEM((HC,), jnp.int32),
            pltpu.VMEM((HC,), jnp.int32),
            pltpu.VMEM((HC, HD), jnp.float32),
            pltpu.VMEM((HC, HD), jnp.float32),
            pltpu.SemaphoreType.DMA,
            pltpu.SemaphoreType.DMA,
            pltpu.SemaphoreType.DMA,
        ],
    )
    def sc_gather_rows(ys_hbm, p_hbm, out_hbm,
                       idx0, idx1, r0, r1, s0, s1, ss):
        wid = lax.axis_index("s") * 2 + lax.axis_index("c")
        base = wid * CHUNK
        row = wid // 2
        col = (wid % 2) * CHUNK
        pltpu.sync_copy(p_hbm.at[row, pl.ds(col, HC)], idx0)
        pltpu.sync_copy(p_hbm.at[row, pl.ds(col + HC, HC)], idx1)
        g0 = pltpu.async_copy(ys_hbm.at[idx0], r0, s0)
        g1 = pltpu.async_copy(ys_hbm.at[idx1], r1, s1)
        g0.wait()
        w0 = pltpu.async_copy(r0, out_hbm.at[pl.ds(base, HC)], ss)
        g1.wait()
        w1 = pltpu.async_copy(r1, out_hbm.at[pl.ds(base + HC, HC)], ss)
        w0.wait()
        w1.wait()

    return sc_scatter_rows, sc_gather_rows


def kernel(x, router_w, router_b, expert_w, expert_b):
    b, s, d = x.shape
    x_flat = x.reshape(s, d)
    p, gid, tid, lo, hi = _router(x_flat, router_w, router_b.reshape(1, NE))
    sc_scatter_rows, sc_gather_rows = _sc_kernels()
    xs = sc_scatter_rows(x_flat, p)
    ys = _gmm(gid, tid, lo, hi, xs, expert_w, expert_b)
    out = sc_gather_rows(ys, p)
    return out.reshape(b, s, HD)
